# baseline (speedup 1.0000x reference)
import functools

import jax
import jax.numpy as jnp
from jax import lax
from jax.experimental import pallas as pl
from jax.experimental.pallas import tpu as pltpu

_NUM_CONV = 6
_NF = 64
_UPSCALE = 4


def _sr_kernel(x_ref, w0_ref, b0_ref, a0_ref, wb_ref, bb_ref, ab_ref,
               wt_ref, bt_ref, e_ref, o_ref, *,
               tile_h, tile_w, halo, num_conv, nf, c_out, r):
    x3 = x_ref[...]
    cin, in_th, in_tw = x3.shape
    w0 = in_tw

    xf = x3.reshape(cin, in_th * in_tw)
    feat = lax.dot_general(xf, w0_ref[...], (((0,), (0,)), ((), ())),
                           preferred_element_type=jnp.float32)
    feat = feat + b0_ref[...]
    feat = jnp.where(feat >= 0.0, feat, feat * a0_ref[...])
    feat = feat.astype(jnp.bfloat16)

    h = in_th
    zpad = jnp.zeros((8, nf), jnp.bfloat16)
    for l in range(num_conv):
        length = h * w0
        lout = (h - 4) * w0
        fpad = jnp.concatenate([feat, zpad], axis=0)
        cols = jnp.concatenate(
            [fpad[dj:dj + length] for dj in range(5)], axis=-1)
        p = jnp.dot(cols, wb_ref[l], preferred_element_type=jnp.float32)
        acc = p[0:lout, 0:nf]
        for di in range(1, 5):
            acc = acc + p[di * w0:di * w0 + lout, di * nf:(di + 1) * nf]
        accb = (acc + bb_ref[l]).astype(jnp.bfloat16)
        feat = jnp.where(accb >= 0, accb, accb * ab_ref[l])
        h = h - 4

    feat = feat.reshape(tile_h, w0, nf)[:, :tile_w, :].reshape(
        tile_h * tile_w, nf)
    y = jnp.dot(feat, wt_ref[...], preferred_element_type=jnp.float32)
    y = y + bt_ref[...]

    base = x3[:, halo:halo + tile_h, halo:halo + tile_w]
    base = base.reshape(cin, tile_h * tile_w)
    y = y + lax.dot_general(base, e_ref[...], (((0,), (0,)), ((), ())),
                            preferred_element_type=jnp.float32)
    y = jnp.clip(y, 0.0, 1.0)
    o_ref[...] = y.reshape(tile_h, tile_w, c_out)


def kernel(x, w_head, b_head, a_head, w_body, b_body, a_body,
           w_tail, b_tail, a_tail):
    num_conv = _NUM_CONV
    nf = _NF
    r = _UPSCALE
    r2 = r * r
    num_in_ch = x.shape[1]
    c_out = num_in_ch * r2
    tile_h, tile_w = 64, 128
    tp = 2 * num_conv

    x = x.astype(jnp.bfloat16)
    n, cin, h_img, w_img = x.shape

    tile_h = h_img if tile_h >= h_img else max(8, (tile_h // 8) * 8)
    tile_w = w_img if tile_w >= w_img else max(8, (tile_w // 8) * 8)
    nth = pl.cdiv(h_img, tile_h)
    ntw = pl.cdiv(w_img, tile_w)
    h_pad, w_pad = nth * tile_h, ntw * tile_w

    halo = tp
    in_th = tile_h + 2 * halo
    in_tw = tile_w + 2 * halo

    x_padded = jnp.pad(
        x, ((0, 0), (0, 0), (tp, tp + h_pad - h_img),
            (tp, tp + w_pad - w_img)))

    tiles = jnp.stack(
        [jnp.stack([x_padded[:, :, i * tile_h:i * tile_h + in_th,
                             j * tile_w:j * tile_w + in_tw]
                    for j in range(ntw)], axis=1)
         for i in range(nth)], axis=1)

    w0 = w_head.reshape(num_in_ch, nf).astype(jnp.bfloat16)
    b0 = b_head.reshape(1, nf).astype(jnp.float32)
    a0 = a_head.reshape(1, nf).astype(jnp.float32)

    wb = jnp.transpose(w_body, (0, 2, 3, 1, 4)).reshape(
        num_conv, 5 * nf, 5 * nf).astype(jnp.bfloat16)
    bb = b_body.reshape(num_conv, 1, nf).astype(jnp.float32)
    ab = a_body.reshape(num_conv, 1, nf).astype(jnp.bfloat16)

    wt = w_tail.reshape(nf, c_out).astype(jnp.bfloat16)
    bt = b_tail.reshape(1, c_out).astype(jnp.float32)

    emat = (jnp.arange(c_out)[None, :] // r2
            == jnp.arange(num_in_ch)[:, None]).astype(jnp.bfloat16)

    kern = functools.partial(
        _sr_kernel, tile_h=tile_h, tile_w=tile_w, halo=halo,
        num_conv=num_conv, nf=nf, c_out=c_out, r=r)

    out_lr = pl.pallas_call(
        kern,
        out_shape=jax.ShapeDtypeStruct((n, h_pad, w_pad, c_out), jnp.float32),
        grid=(n, nth, ntw),
        in_specs=[
            pl.BlockSpec((None, None, None, cin, in_th, in_tw),
                         lambda b, i, j: (b, i, j, 0, 0, 0)),
            pl.BlockSpec(w0.shape, lambda b, i, j: (0, 0)),
            pl.BlockSpec(b0.shape, lambda b, i, j: (0, 0)),
            pl.BlockSpec(a0.shape, lambda b, i, j: (0, 0)),
            pl.BlockSpec(wb.shape, lambda b, i, j: (0, 0, 0)),
            pl.BlockSpec(bb.shape, lambda b, i, j: (0, 0, 0)),
            pl.BlockSpec(ab.shape, lambda b, i, j: (0, 0, 0)),
            pl.BlockSpec(wt.shape, lambda b, i, j: (0, 0)),
            pl.BlockSpec(bt.shape, lambda b, i, j: (0, 0)),
            pl.BlockSpec(emat.shape, lambda b, i, j: (0, 0)),
        ],
        out_specs=pl.BlockSpec((None, tile_h, tile_w, c_out),
                               lambda b, i, j: (b, i, j, 0)),
        compiler_params=pltpu.CompilerParams(
            dimension_semantics=("parallel", "parallel", "parallel"),
            vmem_limit_bytes=60 * 1024 * 1024),
    )(tiles, w0, b0, a0, wb, bb, ab, wt, bt, emat)

    out = out_lr[:, :h_img, :w_img, :]
    nb, hb, wb_, cb = out.shape
    cfin = cb // r2
    out = out.reshape(nb, hb, wb_, cfin, r, r)
    out = jnp.transpose(out, (0, 1, 4, 2, 5, 3))
    out = out.reshape(nb, hb * r, wb_ * r, cfin)
    return jnp.transpose(out, (0, 3, 1, 2))


# speedup vs baseline: 1.0233x; 1.0233x over previous
import functools

import jax
import jax.numpy as jnp
from jax import lax
from jax.experimental import pallas as pl
from jax.experimental.pallas import tpu as pltpu

_NUM_CONV = 6
_NF = 64
_UPSCALE = 4


def _sr_kernel(x_ref, w0_ref, b0_ref, a0_ref, wb_ref, bb_ref, ab_ref,
               wt_ref, bt_ref, e_ref, o_ref, *,
               tile_h, tile_w, halo, num_conv, nf, c_out, r):
    x3 = x_ref[...]
    cin, in_th, in_tw = x3.shape
    w0 = in_tw

    xf = x3.reshape(cin, in_th * in_tw)
    feat = lax.dot_general(xf, w0_ref[...], (((0,), (0,)), ((), ())),
                           preferred_element_type=jnp.float32)
    feat = feat + b0_ref[...]
    feat = jnp.where(feat >= 0.0, feat, feat * a0_ref[...])
    feat = feat.astype(jnp.bfloat16)

    h = in_th
    zpad = jnp.zeros((8, nf), jnp.bfloat16)
    for l in range(num_conv):
        length = h * w0
        lout = (h - 4) * w0
        fpad = jnp.concatenate([feat, zpad], axis=0)
        hh = ((h - 4) // 2 // 8) * 8
        outs = []
        for (r0, ro) in ((0, hh), (hh, h - 4 - hh)):
            lo = r0 * w0
            lc = (ro + 4) * w0
            cols = jnp.concatenate(
                [fpad[lo + dj:lo + dj + lc] for dj in range(5)], axis=-1)
            p = jnp.dot(cols, wb_ref[l], preferred_element_type=jnp.float32)
            acc = p[0:ro * w0, 0:nf]
            for di in range(1, 5):
                acc = acc + p[di * w0:di * w0 + ro * w0,
                              di * nf:(di + 1) * nf]
            accb = (acc + bb_ref[l]).astype(jnp.bfloat16)
            outs.append(jnp.where(accb >= 0, accb, accb * ab_ref[l]))
        feat = jnp.concatenate(outs, axis=0)
        h = h - 4

    feat = feat.reshape(tile_h, w0, nf)[:, :tile_w, :].reshape(
        tile_h * tile_w, nf)
    y = jnp.dot(feat, wt_ref[...], preferred_element_type=jnp.float32)
    y = y + bt_ref[...]

    base = x3[:, halo:halo + tile_h, halo:halo + tile_w]
    base = base.reshape(cin, tile_h * tile_w)
    y = y + lax.dot_general(base, e_ref[...], (((0,), (0,)), ((), ())),
                            preferred_element_type=jnp.float32)
    y = jnp.clip(y, 0.0, 1.0)
    o_ref[...] = y.reshape(tile_h, tile_w, c_out)


def kernel(x, w_head, b_head, a_head, w_body, b_body, a_body,
           w_tail, b_tail, a_tail):
    num_conv = _NUM_CONV
    nf = _NF
    r = _UPSCALE
    r2 = r * r
    num_in_ch = x.shape[1]
    c_out = num_in_ch * r2
    tile_h, tile_w = 64, 128
    tp = 2 * num_conv

    x = x.astype(jnp.float32)
    n, cin, h_img, w_img = x.shape

    tile_h = h_img if tile_h >= h_img else max(8, (tile_h // 8) * 8)
    tile_w = w_img if tile_w >= w_img else max(8, (tile_w // 8) * 8)
    nth = pl.cdiv(h_img, tile_h)
    ntw = pl.cdiv(w_img, tile_w)
    h_pad, w_pad = nth * tile_h, ntw * tile_w

    halo = tp
    in_th = tile_h + 2 * halo
    in_tw = tile_w + 2 * halo

    x_padded = jnp.pad(
        x, ((0, 0), (0, 0), (tp, tp + h_pad - h_img),
            (tp, tp + w_pad - w_img)))

    tiles = jnp.stack(
        [jnp.stack([x_padded[:, :, i * tile_h:i * tile_h + in_th,
                             j * tile_w:j * tile_w + in_tw]
                    for j in range(ntw)], axis=1)
         for i in range(nth)], axis=1)

    w0 = w_head.reshape(num_in_ch, nf).astype(jnp.float32)
    b0 = b_head.reshape(1, nf).astype(jnp.float32)
    a0 = a_head.reshape(1, nf).astype(jnp.float32)

    wb = jnp.transpose(w_body, (0, 2, 3, 1, 4)).reshape(
        num_conv, 5 * nf, 5 * nf).astype(jnp.bfloat16)
    bb = b_body.reshape(num_conv, 1, nf).astype(jnp.float32)
    ab = a_body.reshape(num_conv, 1, nf).astype(jnp.bfloat16)

    wt = w_tail.reshape(nf, c_out).astype(jnp.bfloat16)
    bt = b_tail.reshape(1, c_out).astype(jnp.float32)

    emat = (jnp.arange(c_out)[None, :] // r2
            == jnp.arange(num_in_ch)[:, None]).astype(jnp.float32)

    kern = functools.partial(
        _sr_kernel, tile_h=tile_h, tile_w=tile_w, halo=halo,
        num_conv=num_conv, nf=nf, c_out=c_out, r=r)

    out_lr = pl.pallas_call(
        kern,
        out_shape=jax.ShapeDtypeStruct((n, h_pad, w_pad, c_out), jnp.float32),
        grid=(n, nth, ntw),
        in_specs=[
            pl.BlockSpec((None, None, None, cin, in_th, in_tw),
                         lambda b, i, j: (b, i, j, 0, 0, 0)),
            pl.BlockSpec(w0.shape, lambda b, i, j: (0, 0)),
            pl.BlockSpec(b0.shape, lambda b, i, j: (0, 0)),
            pl.BlockSpec(a0.shape, lambda b, i, j: (0, 0)),
            pl.BlockSpec(wb.shape, lambda b, i, j: (0, 0, 0)),
            pl.BlockSpec(bb.shape, lambda b, i, j: (0, 0, 0)),
            pl.BlockSpec(ab.shape, lambda b, i, j: (0, 0, 0)),
            pl.BlockSpec(wt.shape, lambda b, i, j: (0, 0)),
            pl.BlockSpec(bt.shape, lambda b, i, j: (0, 0)),
            pl.BlockSpec(emat.shape, lambda b, i, j: (0, 0)),
        ],
        out_specs=pl.BlockSpec((None, tile_h, tile_w, c_out),
                               lambda b, i, j: (b, i, j, 0)),
        compiler_params=pltpu.CompilerParams(
            dimension_semantics=("parallel", "parallel", "parallel"),
            vmem_limit_bytes=60 * 1024 * 1024),
    )(tiles, w0, b0, a0, wb, bb, ab, wt, bt, emat)

    out = out_lr[:, :h_img, :w_img, :]
    nb, hb, wb_, cb = out.shape
    cfin = cb // r2
    out = out.reshape(nb, hb, wb_, cfin, r, r)
    out = jnp.transpose(out, (0, 1, 4, 2, 5, 3))
    out = out.reshape(nb, hb * r, wb_ * r, cfin)
    return jnp.transpose(out, (0, 3, 1, 2))
